# baseline (speedup 1.0000x reference)
import functools

import jax
import jax.numpy as jnp
from jax import lax
from jax.experimental import pallas as pl
from jax.experimental.pallas import tpu as pltpu
from jax.experimental.pallas import tpu_sc as plsc

L = 16
NFEAT = 100
NPAD = 112
K = 16
NH0 = 10
D0 = NFEAT * K


def _deepfm_body(idx_hbm, w2d_hbm, v_hbm, w0m_hbm, smalls_hbm, out_hbm,
                 idx_v, V_v, W0_v, smalls_v, res_v, wg_v,
                 sem_v, sem_w, sem_w0, sem_s):
    cid = lax.axis_index("c")
    sid = lax.axis_index("s")

    @pl.when(jnp.logical_and(cid == 0, sid == 0))
    def _():
        cp_w0 = pltpu.async_copy(w0m_hbm, W0_v, sem_w0)
        cp_s = pltpu.async_copy(smalls_hbm, smalls_v, sem_s)
        zero = jnp.zeros((L,), jnp.float32)
        wg_v[NFEAT // L, :] = zero
        pltpu.sync_copy(idx_hbm, idx_v)
        copies = []
        for t in range(NPAD // L):
            chunk = idx_v[pl.ds(t * L, L)]
            for l in range(L):
                c = t * L + l
                if c >= NFEAT:
                    break
                r = chunk[l]
                copies.append(
                    pltpu.async_copy(v_hbm.at[r], V_v.at[c], sem_v))
                copies.append(
                    pltpu.async_copy(w2d_hbm.at[r], wg_v.at[t, pl.ds(l, 1)],
                                     sem_w))
        cp_w0.wait()
        cp_s.wait()
        for cp in copies:
            cp.wait()

        def loop_body(c, carry):
            s = carry[0]
            q = carry[1]
            v = V_v[c, :]
            base = c * K
            new_acc = tuple(carry[2 + j] + v * W0_v[j, pl.ds(base, K)]
                            for j in range(NH0))
            return (s + v, q + v * v) + new_acc

        init = (zero, zero) + (zero,) * NH0
        fin = lax.fori_loop(0, NFEAT, loop_body, init)
        s, q = fin[0], fin[1]
        acc = fin[2:]

        linv = zero
        for t in range(NPAD // L):
            linv = linv + wg_v[t, :]
        lin = jnp.sum(linv)

        lanes = lax.iota(jnp.int32, L)
        row_b0 = smalls_v[0, :]
        h0 = zero
        for j in range(NH0):
            d = jnp.maximum(jnp.sum(acc[j]) + row_b0[j], 0.0)
            h0 = h0 + jnp.where(lanes == j, d, 0.0)
        row_b1 = smalls_v[6, :]
        h1 = zero
        for j in range(5):
            d = jnp.maximum(jnp.sum(h0 * smalls_v[1 + j, :]) + row_b1[j], 0.0)
            h1 = h1 + jnp.where(lanes == j, d, 0.0)
        row_b2 = smalls_v[10, :]
        h2 = zero
        for j in range(3):
            d = jnp.maximum(jnp.sum(h1 * smalls_v[7 + j, :]) + row_b2[j], 0.0)
            h2 = h2 + jnp.where(lanes == j, d, 0.0)
        row_w3 = smalls_v[11, :]
        dnn = jnp.sum(h2 * row_w3)
        b3s = row_w3[3]
        w0s = row_w3[4]

        pair = 0.5 * (jnp.sum(s * s) - jnp.sum(q))
        res = pair + lin + w0s + dnn + b3s
        res_v[:] = jnp.where(lanes == 0, res, 0.0)
        pltpu.sync_copy(res_v.at[pl.ds(0, 1)], out_hbm)


_deepfm_sc = functools.partial(
    pl.kernel,
    out_type=jax.ShapeDtypeStruct((1,), jnp.float32),
    mesh=plsc.VectorSubcoreMesh(core_axis_name="c", subcore_axis_name="s"),
    compiler_params=pltpu.CompilerParams(needs_layout_passes=False),
    scratch_types=[
        pltpu.VMEM((NPAD,), jnp.int32),
        pltpu.VMEM((NPAD, K), jnp.float32),
        pltpu.VMEM((NH0, D0), jnp.float32),
        pltpu.VMEM((12, L), jnp.float32),
        pltpu.VMEM((L,), jnp.float32),
        pltpu.VMEM((NPAD // L, L), jnp.float32),
        pltpu.SemaphoreType.DMA,
        pltpu.SemaphoreType.DMA,
        pltpu.SemaphoreType.DMA,
        pltpu.SemaphoreType.DMA,
    ],
)(_deepfm_body)


def kernel(feature, w_table, v_table, w0, W0, b0, W1, b1, W2, b2, W3, b3):
    feature = feature.astype(jnp.int32)
    idx = jnp.concatenate([feature, jnp.zeros((NPAD - NFEAT,), jnp.int32)])
    row_b0 = jnp.pad(b0, (0, L - NH0))
    w1_rows = jnp.pad(W1, ((0, 0), (0, L - NH0)))
    row_b1 = jnp.pad(b1, (0, L - 5))
    w2_rows = jnp.pad(W2, ((0, 0), (0, L - 5)))
    row_b2 = jnp.pad(b2, (0, L - 3))
    row_w3 = jnp.concatenate([W3[0], b3, w0, jnp.zeros((L - 5,), jnp.float32)])
    smalls = jnp.concatenate(
        [row_b0[None], w1_rows, row_b1[None], w2_rows, row_b2[None],
         row_w3[None]], axis=0)
    return _deepfm_sc(idx, w_table, v_table, W0, smalls)


# speedup vs baseline: 6.1197x; 6.1197x over previous
import functools

import jax
import jax.numpy as jnp
from jax import lax
from jax.experimental import pallas as pl
from jax.experimental.pallas import tpu as pltpu
from jax.experimental.pallas import tpu_sc as plsc

L = 16
NFEAT = 100
NPAD = 112
K = 16
NH0 = 10
D0 = NFEAT * K
TILE = 128
DEPTH = 32


def _deepfm_body(idx_hbm, w1d_hbm, vT_hbm, w0m_hbm, smalls_hbm, out_hbm,
                 idx_v, blk_v, V_v, wblk_v, W0_v, smalls_v, res_v,
                 sem_v, sem_w, sem_w0, sem_s):
    cid = lax.axis_index("c")
    sid = lax.axis_index("s")

    @pl.when(jnp.logical_and(cid == 0, sid == 0))
    def _():
        cp_w0 = pltpu.async_copy(w0m_hbm, W0_v, sem_w0)
        cp_s = pltpu.async_copy(smalls_hbm, smalls_v, sem_s)
        pltpu.sync_copy(idx_hbm, idx_v)

        lanes = lax.iota(jnp.int32, L)
        zero = jnp.zeros((L,), jnp.float32)

        v_copies = [None] * NFEAT
        w_copies = [None] * NFEAT
        offs = [None] * NFEAT
        lin = jnp.float32(0.0)

        def extract(c):
            v_copies[c].wait()
            w_copies[c].wait()
            slot = jnp.full((L,), c % DEPTH, jnp.int32)
            off = jnp.full((L,), offs[c], jnp.int32)
            v = plsc.load_gather(blk_v, [slot, lanes, off])
            V_v[c, :] = v
            wrow = jnp.full((L,), c, jnp.int32)
            wv = plsc.load_gather(wblk_v, [wrow, off])
            return wv[0]

        chunk = None
        for c in range(NFEAT):
            t, l = divmod(c, L)
            if l == 0:
                chunk = idx_v[pl.ds(t * L, L)]
            r = chunk[l]
            rb = r // TILE
            base = pl.multiple_of(rb * TILE, TILE)
            offs[c] = r - base
            if c >= DEPTH:
                lin = lin + extract(c - DEPTH)
            v_copies[c] = pltpu.async_copy(
                vT_hbm.at[:, pl.ds(base, TILE)], blk_v.at[c % DEPTH], sem_v)
            w_copies[c] = pltpu.async_copy(
                w1d_hbm.at[pl.ds(base, TILE)], wblk_v.at[c], sem_w)
        for c in range(max(NFEAT - DEPTH, 0), NFEAT):
            lin = lin + extract(c)
        cp_w0.wait()
        cp_s.wait()

        def loop_body(c, carry):
            s = carry[0]
            q = carry[1]
            v = V_v[c, :]
            base = c * K
            new_acc = tuple(carry[2 + j] + v * W0_v[j, pl.ds(base, K)]
                            for j in range(NH0))
            return (s + v, q + v * v) + new_acc

        init = (zero, zero) + (zero,) * NH0
        fin = lax.fori_loop(0, NFEAT, loop_body, init)
        s, q = fin[0], fin[1]
        acc = fin[2:]

        row_b0 = smalls_v[0, :]
        h0 = zero
        for j in range(NH0):
            d = jnp.maximum(jnp.sum(acc[j]) + row_b0[j], 0.0)
            h0 = h0 + jnp.where(lanes == j, d, 0.0)
        row_b1 = smalls_v[6, :]
        h1 = zero
        for j in range(5):
            d = jnp.maximum(jnp.sum(h0 * smalls_v[1 + j, :]) + row_b1[j], 0.0)
            h1 = h1 + jnp.where(lanes == j, d, 0.0)
        row_b2 = smalls_v[10, :]
        h2 = zero
        for j in range(3):
            d = jnp.maximum(jnp.sum(h1 * smalls_v[7 + j, :]) + row_b2[j], 0.0)
            h2 = h2 + jnp.where(lanes == j, d, 0.0)
        row_w3 = smalls_v[11, :]
        dnn = jnp.sum(h2 * row_w3)
        b3s = row_w3[3]
        w0s = row_w3[4]

        pair = 0.5 * (jnp.sum(s * s) - jnp.sum(q))
        res = pair + lin + w0s + dnn + b3s
        res_v[:] = jnp.where(lanes == 0, res, 0.0)
        pltpu.sync_copy(res_v.at[pl.ds(0, 1)], out_hbm)


_deepfm_sc = functools.partial(
    pl.kernel,
    out_type=jax.ShapeDtypeStruct((1,), jnp.float32),
    mesh=plsc.VectorSubcoreMesh(core_axis_name="c", subcore_axis_name="s"),
    compiler_params=pltpu.CompilerParams(needs_layout_passes=False,
                                         use_tc_tiling_on_sc=True),
    scratch_types=[
        pltpu.VMEM((NPAD,), jnp.int32),
        pltpu.VMEM((DEPTH, K, TILE), jnp.float32),
        pltpu.VMEM((NPAD, K), jnp.float32),
        pltpu.VMEM((NFEAT, TILE), jnp.float32),
        pltpu.VMEM((NH0, D0), jnp.float32),
        pltpu.VMEM((12, L), jnp.float32),
        pltpu.VMEM((L,), jnp.float32),
        pltpu.SemaphoreType.DMA,
        pltpu.SemaphoreType.DMA,
        pltpu.SemaphoreType.DMA,
        pltpu.SemaphoreType.DMA,
    ],
)(_deepfm_body)


def kernel(feature, w_table, v_table, w0, W0, b0, W1, b1, W2, b2, W3, b3):
    feature = feature.astype(jnp.int32)
    idx = jnp.concatenate([feature, jnp.zeros((NPAD - NFEAT,), jnp.int32)])
    row_b0 = jnp.pad(b0, (0, L - NH0))
    w1_rows = jnp.pad(W1, ((0, 0), (0, L - NH0)))
    row_b1 = jnp.pad(b1, (0, L - 5))
    w2_rows = jnp.pad(W2, ((0, 0), (0, L - 5)))
    row_b2 = jnp.pad(b2, (0, L - 3))
    row_w3 = jnp.concatenate([W3[0], b3, w0, jnp.zeros((L - 5,), jnp.float32)])
    smalls = jnp.concatenate(
        [row_b0[None], w1_rows, row_b1[None], w2_rows, row_b2[None],
         row_w3[None]], axis=0)
    return _deepfm_sc(idx, w_table.reshape(-1), v_table.T, W0, smalls)


# speedup vs baseline: 13.9154x; 2.2739x over previous
import functools

import jax
import jax.numpy as jnp
from jax import lax
from jax.experimental import pallas as pl
from jax.experimental.pallas import tpu as pltpu
from jax.experimental.pallas import tpu_sc as plsc

L = 16
NFEAT = 100
NPAD = 112
K = 16
NH0 = 10
D0 = NFEAT * K
TILE = 128
DEPTH = 32


def _deepfm_body(idx_hbm, wT_hbm, vT_hbm, w0m_hbm, smalls_hbm, out_hbm,
                 idx_v, blk_v, V_v, wblk_v, W0_v, smalls_v, res_v,
                 sem_v, sem_w, sem_w0, sem_s):
    cid = lax.axis_index("c")
    sid = lax.axis_index("s")

    @pl.when(jnp.logical_and(cid == 0, sid == 0))
    def _():
        cp_w0 = pltpu.async_copy(w0m_hbm, W0_v, sem_w0)
        cp_s = pltpu.async_copy(smalls_hbm, smalls_v, sem_s)
        pltpu.sync_copy(idx_hbm, idx_v.at[pl.ds(0, NFEAT)])

        lanes = lax.iota(jnp.int32, L)
        zero = jnp.zeros((L,), jnp.float32)

        v_copies = [None] * NFEAT
        w_copies = [None] * NFEAT
        offs = [None] * NFEAT
        lin = jnp.float32(0.0)

        def extract(c):
            v_copies[c].wait()
            w_copies[c].wait()
            slot = jnp.full((L,), c % DEPTH, jnp.int32)
            off = jnp.full((L,), offs[c], jnp.int32)
            v = plsc.load_gather(blk_v, [slot, lanes, off])
            V_v[c, :] = v
            wv = plsc.load_gather(
                wblk_v, [jnp.full((L,), c, jnp.int32),
                         jnp.zeros((L,), jnp.int32), off])
            return wv[0]

        chunk = None
        for c in range(NFEAT):
            t, l = divmod(c, L)
            if l == 0:
                chunk = idx_v[pl.ds(t * L, L)]
            r = chunk[l]
            rb = r // TILE
            base = pl.multiple_of(rb * TILE, TILE)
            offs[c] = r - base
            if c >= DEPTH:
                lin = lin + extract(c - DEPTH)
            v_copies[c] = pltpu.async_copy(
                vT_hbm.at[:, pl.ds(base, TILE)], blk_v.at[c % DEPTH], sem_v)
            w_copies[c] = pltpu.async_copy(
                wT_hbm.at[:, pl.ds(base, TILE)], wblk_v.at[c], sem_w)
        for c in range(max(NFEAT - DEPTH, 0), NFEAT):
            lin = lin + extract(c)
        cp_w0.wait()

        def loop_body(c, carry):
            s = carry[0]
            q = carry[1]
            v = V_v[c, :]
            base = c * K
            new_acc = tuple(carry[2 + j] + v * W0_v[j, pl.ds(base, K)]
                            for j in range(NH0))
            return (s + v, q + v * v) + new_acc

        init = (zero, zero) + (zero,) * NH0
        fin = lax.fori_loop(0, NFEAT, loop_body, init)
        s, q = fin[0], fin[1]
        acc = fin[2:]

        cp_s.wait()

        def srow(off):
            return plsc.load_gather(smalls_v, [lanes + off])

        row_b0 = srow(0)
        h0 = zero
        for j in range(NH0):
            d = jnp.maximum(jnp.sum(acc[j]) + row_b0[j], 0.0)
            h0 = h0 + jnp.where(lanes == j, d, 0.0)
        row_b1 = srow(60)
        h1 = zero
        for j in range(5):
            d = jnp.maximum(jnp.sum(h0 * srow(10 + NH0 * j)) + row_b1[j], 0.0)
            h1 = h1 + jnp.where(lanes == j, d, 0.0)
        row_b2 = srow(80)
        h2 = zero
        for j in range(3):
            d = jnp.maximum(jnp.sum(h1 * srow(65 + 5 * j)) + row_b2[j], 0.0)
            h2 = h2 + jnp.where(lanes == j, d, 0.0)
        dnn = jnp.sum(h2 * srow(83))
        tailv = srow(86)
        b3s = tailv[0]
        w0s = tailv[1]

        pair = 0.5 * (jnp.sum(s * s) - jnp.sum(q))
        res = pair + lin + w0s + dnn + b3s
        res_v[:] = jnp.where(lanes == 0, res, 0.0)
        pltpu.sync_copy(res_v.at[pl.ds(0, 1)], out_hbm)


_deepfm_sc = functools.partial(
    pl.kernel,
    out_type=jax.ShapeDtypeStruct((1,), jnp.float32),
    mesh=plsc.VectorSubcoreMesh(core_axis_name="c", subcore_axis_name="s"),
    compiler_params=pltpu.CompilerParams(needs_layout_passes=False,
                                         use_tc_tiling_on_sc=True),
    scratch_types=[
        pltpu.VMEM((NPAD,), jnp.int32),
        pltpu.VMEM((DEPTH, K, TILE), jnp.float32),
        pltpu.VMEM((NPAD, K), jnp.float32),
        pltpu.VMEM((NFEAT, 1, TILE), jnp.float32),
        pltpu.VMEM((NH0, D0), jnp.float32),
        pltpu.VMEM((NPAD,), jnp.float32),
        pltpu.VMEM((L,), jnp.float32),
        pltpu.SemaphoreType.DMA,
        pltpu.SemaphoreType.DMA,
        pltpu.SemaphoreType.DMA,
        pltpu.SemaphoreType.DMA,
    ],
)(_deepfm_body)


def kernel(feature, w_table, v_table, w0, W0, b0, W1, b1, W2, b2, W3, b3):
    smalls = jnp.concatenate(
        [b0, W1.ravel(), b1, W2.ravel(), b2, W3.ravel(), b3, w0,
         jnp.zeros((NPAD - 88,), jnp.float32)])
    return _deepfm_sc(feature.astype(jnp.int32), w_table.T, v_table.T, W0,
                      smalls)


# speedup vs baseline: 18.7692x; 1.3488x over previous
import functools

import jax
import jax.numpy as jnp
from jax import lax
from jax.experimental import pallas as pl
from jax.experimental.pallas import tpu as pltpu
from jax.experimental.pallas import tpu_sc as plsc

L = 16
NFEAT = 100
NPAD = 112
K = 16
NH0 = 10
D0 = NFEAT * K
TILE = 128
FPT = 8
NT_G = 13


def _deepfm_body(idx_hbm, wT_hbm, vT_hbm, w0m_hbm, smalls_hbm, out_hbm,
                 idx_v, blk_v, wblk_v, Vloc_v, part_v, pall_v, Vall_v, W0_v,
                 smalls_v, res_v, V_sh, parts_sh,
                 sem_v, sem_w, sem_w0, sem_s):
    cid = lax.axis_index("c")
    sid = lax.axis_index("s")

    @pl.when(cid == 0)
    def _():
        lanes = lax.iota(jnp.int32, L)
        zero = jnp.zeros((L,), jnp.float32)

        cp_w0 = pltpu.async_copy(w0m_hbm, W0_v, sem_w0)
        cp_s = pltpu.async_copy(smalls_hbm, smalls_v, sem_s)
        pltpu.sync_copy(idx_hbm, idx_v.at[pl.ds(0, NFEAT)])

        sbase = jnp.minimum(sid, NT_G - 1) * FPT
        chunk = plsc.load_gather(idx_v, [lanes + sbase])

        rs, offs, valids = [], [], []
        for l in range(FPT):
            valid = sid * FPT + l < NFEAT
            r = jnp.where(valid, chunk[l], 0)
            rb = r // TILE
            base = pl.multiple_of(rb * TILE, TILE)
            pltpu.async_copy(vT_hbm.at[:, pl.ds(base, TILE)], blk_v.at[l],
                             sem_v)
            pltpu.async_copy(wT_hbm.at[:, pl.ds(base, TILE)], wblk_v.at[l],
                             sem_w)
            offs.append(r - base)
            valids.append(valid)

        for l in range(FPT):
            pltpu.make_async_copy(vT_hbm.at[:, pl.ds(0, TILE)], blk_v.at[l],
                                  sem_v).wait()
            pltpu.make_async_copy(wT_hbm.at[:, pl.ds(0, TILE)], wblk_v.at[l],
                                  sem_w).wait()
        s_part = zero
        q_part = zero
        lin = jnp.float32(0.0)
        for l in range(FPT):
            scale = jnp.where(valids[l], jnp.float32(1.0), jnp.float32(0.0))
            off = jnp.full((L,), offs[l], jnp.int32)
            v = plsc.load_gather(
                blk_v, [jnp.full((L,), l, jnp.int32), lanes, off]) * scale
            Vloc_v[l, :] = v
            s_part = s_part + v
            q_part = q_part + v * v
            wv = plsc.load_gather(
                wblk_v, [jnp.full((L,), l, jnp.int32),
                         jnp.zeros((L,), jnp.int32), off])
            lin = lin + wv[0] * scale
        part_v[0, :] = s_part
        part_v[1, :] = q_part
        part_v[2, :] = jnp.where(lanes == 0, lin, 0.0)
        pltpu.sync_copy(part_v.at[pl.ds(0, 3)],
                        parts_sh.at[sid, pl.ds(0, 3)])

        @pl.when(sid < NT_G)
        def _publish_rows():
            pltpu.sync_copy(Vloc_v, V_sh.at[pl.ds(sid * FPT, FPT)])

        cp_w0.wait()
        plsc.subcore_barrier()

        pltpu.sync_copy(V_sh, Vall_v)
        j = jnp.minimum(sid, NH0 - 1)

        def loop_body(c, accj):
            return accj + Vall_v[c, :] * W0_v[j, pl.ds(c * K, K)]

        accj = lax.fori_loop(0, NFEAT, loop_body, zero)
        dj = jnp.sum(accj)
        part_v[3, :] = jnp.where(lanes == sid, dj, 0.0)
        pltpu.sync_copy(part_v.at[pl.ds(3, 1)], parts_sh.at[sid, pl.ds(3, 1)])
        cp_s.wait()

        plsc.subcore_barrier()

        @pl.when(sid == 0)
        def _tail():
            pltpu.sync_copy(parts_sh, pall_v)
            s = zero
            q = zero
            linv = zero
            dnn0 = zero
            for t in range(L):
                s = s + pall_v[t, 0, :]
                q = q + pall_v[t, 1, :]
                linv = linv + pall_v[t, 2, :]
                dnn0 = dnn0 + pall_v[t, 3, :]
            lin_t = linv[0]

            def srow(off):
                return plsc.load_gather(smalls_v, [lanes + off])

            row_b0 = srow(0)
            h0 = zero
            for jj in range(NH0):
                d = jnp.maximum(dnn0[jj] + row_b0[jj], 0.0)
                h0 = h0 + jnp.where(lanes == jj, d, 0.0)
            row_b1 = srow(60)
            h1 = zero
            for jj in range(5):
                d = jnp.maximum(jnp.sum(h0 * srow(10 + NH0 * jj))
                                + row_b1[jj], 0.0)
                h1 = h1 + jnp.where(lanes == jj, d, 0.0)
            row_b2 = srow(80)
            h2 = zero
            for jj in range(3):
                d = jnp.maximum(jnp.sum(h1 * srow(65 + 5 * jj))
                                + row_b2[jj], 0.0)
                h2 = h2 + jnp.where(lanes == jj, d, 0.0)
            dnn = jnp.sum(h2 * srow(83))
            tailv = srow(86)

            pair = 0.5 * (jnp.sum(s * s) - jnp.sum(q))
            res = pair + lin_t + tailv[1] + dnn + tailv[0]
            res_v[:] = jnp.where(lanes == 0, res, 0.0)
            pltpu.sync_copy(res_v.at[pl.ds(0, 1)], out_hbm)


_deepfm_sc = functools.partial(
    pl.kernel,
    out_type=jax.ShapeDtypeStruct((1,), jnp.float32),
    mesh=plsc.VectorSubcoreMesh(core_axis_name="c", subcore_axis_name="s"),
    compiler_params=pltpu.CompilerParams(needs_layout_passes=False,
                                         use_tc_tiling_on_sc=True),
    scratch_types=[
        pltpu.VMEM((NPAD,), jnp.int32),
        pltpu.VMEM((FPT, K, TILE), jnp.float32),
        pltpu.VMEM((FPT, 1, TILE), jnp.float32),
        pltpu.VMEM((FPT, K), jnp.float32),
        pltpu.VMEM((4, L), jnp.float32),
        pltpu.VMEM((L, 4, L), jnp.float32),
        pltpu.VMEM((NPAD, K), jnp.float32),
        pltpu.VMEM((NH0, D0), jnp.float32),
        pltpu.VMEM((NPAD,), jnp.float32),
        pltpu.VMEM((L,), jnp.float32),
        pltpu.VMEM_SHARED((NPAD, K), jnp.float32),
        pltpu.VMEM_SHARED((L, 4, L), jnp.float32),
        pltpu.SemaphoreType.DMA,
        pltpu.SemaphoreType.DMA,
        pltpu.SemaphoreType.DMA,
        pltpu.SemaphoreType.DMA,
    ],
)(_deepfm_body)


def kernel(feature, w_table, v_table, w0, W0, b0, W1, b1, W2, b2, W3, b3):
    smalls = jnp.concatenate(
        [b0, W1.ravel(), b1, W2.ravel(), b2, W3.ravel(), b3, w0,
         jnp.zeros((NPAD - 88,), jnp.float32)])
    return _deepfm_sc(feature.astype(jnp.int32), w_table.T, v_table.T, W0,
                      smalls)
